# baseline (speedup 1.0000x reference)
import functools

import jax
import jax.numpy as jnp
from jax import lax
from jax.experimental import pallas as pl
from jax.experimental.pallas import tpu as pltpu
from jax.experimental.pallas import tpu_sc as plsc

_BATCH = 16384
_SEQ = 200
_DIM = 32
_VOCAB = 1000
_N_TOTAL = _BATCH * _SEQ
_NUM_CORES = 2
_NUM_SUBCORES = 16
_NW = _NUM_CORES * _NUM_SUBCORES
_B_PER_W = _N_TOTAL // _NW
_CHUNK = 1280
_N_CHUNKS = _B_PER_W // _CHUNK
_GROUPS = _CHUNK // 16
_LANES = 16

_mesh = plsc.VectorSubcoreMesh(core_axis_name="c", subcore_axis_name="s")


@functools.partial(
    pl.kernel,
    mesh=_mesh,
    out_type=jax.ShapeDtypeStruct((_N_TOTAL * _DIM,), jnp.float32),
    scratch_types=[
        pltpu.VMEM((_CHUNK,), jnp.int32),
        pltpu.VMEM((_CHUNK,), jnp.int32),
        pltpu.VMEM((_CHUNK * _DIM,), jnp.float32),
        pltpu.VMEM((_CHUNK * _DIM,), jnp.float32),
        pltpu.VMEM((_VOCAB * _DIM,), jnp.float32),
        pltpu.SemaphoreType.DMA,
        pltpu.SemaphoreType.DMA,
        pltpu.SemaphoreType.DMA,
        pltpu.SemaphoreType.DMA,
    ],
    compiler_params=pltpu.CompilerParams(use_tc_tiling_on_sc=False,
                                         needs_layout_passes=False),
)
def _gather_kernel(idx_hbm, table_hbm, out_hbm,
                   idx0, idx1, rows0, rows1, table_v,
                   sl0, sl1, ss0, ss1):
    sid = lax.axis_index("s")
    wid = sid * _NUM_CORES + lax.axis_index("c")
    base = wid * _B_PER_W

    idx = (idx0, idx1)
    rows = (rows0, rows1)
    sl = (sl0, sl1)
    ss = (ss0, ss1)

    pltpu.sync_copy(table_hbm, table_v)

    lane = lax.iota(jnp.int32, _LANES)
    lane32 = lane * _DIM

    def issue_l(i, b):
        pltpu.async_copy(idx_hbm.at[pl.ds(base + i * _CHUNK, _CHUNK)],
                         idx[b], sl[b])

    def wait_l(b):
        pltpu.make_async_copy(idx_hbm.at[pl.ds(base, _CHUNK)],
                              idx[b], sl[b]).wait()

    def issue_s(i, b):
        pltpu.async_copy(
            rows[b],
            out_hbm.at[pl.ds((base + i * _CHUNK) * _DIM, _CHUNK * _DIM)],
            ss[b])

    def wait_s(b):
        pltpu.make_async_copy(rows[b],
                              out_hbm.at[pl.ds(base * _DIM, _CHUNK * _DIM)],
                              ss[b]).wait()

    def compute(b):
        idx_ref = idx[b]
        rows_ref = rows[b]

        def group(g, carry):
            idx16 = idx_ref[pl.ds(g * _LANES, _LANES)]
            src = idx16 * _DIM
            dst = g * (_LANES * _DIM) + lane32
            for d in range(_DIM):
                vals = plsc.load_gather(table_v, [src + d])
                plsc.store_scatter(rows_ref, [dst + d], vals)
            return carry

        lax.fori_loop(0, _GROUPS, group, 0)

    issue_l(0, 0)

    def step(i, b, ob):
        @pl.when(i >= 2)
        def _():
            wait_s(b)

        wait_l(b)

        @pl.when(i + 1 < _N_CHUNKS)
        def _():
            issue_l(i + 1, ob)

        compute(b)
        issue_s(i, b)

    def outer(g, carry):
        step(2 * g, 0, 1)
        step(2 * g + 1, 1, 0)
        return carry

    lax.fori_loop(0, _N_CHUNKS // 2, outer, 0)

    wait_s(0)
    wait_s(1)


def kernel(sentence, table):
    flat_idx = sentence.reshape(_N_TOTAL)
    out = _gather_kernel(flat_idx, table.reshape(_VOCAB * _DIM))
    return out.reshape(_BATCH, _SEQ, _DIM)


# speedup vs baseline: 1.2090x; 1.2090x over previous
import functools

import jax
import jax.numpy as jnp
from jax import lax
from jax.experimental import pallas as pl
from jax.experimental.pallas import tpu as pltpu
from jax.experimental.pallas import tpu_sc as plsc

_BATCH = 16384
_SEQ = 200
_DIM = 32
_VOCAB = 1000
_N_TOTAL = _BATCH * _SEQ
_NUM_CORES = 2
_NUM_SUBCORES = 16
_NW = _NUM_CORES * _NUM_SUBCORES
_B_PER_W = _N_TOTAL // _NW
_CHUNK = 1280
_N_CHUNKS = _B_PER_W // _CHUNK
_GROUPS = _CHUNK // 16
_LANES = 16

_mesh = plsc.VectorSubcoreMesh(core_axis_name="c", subcore_axis_name="s")


@functools.partial(
    pl.kernel,
    mesh=_mesh,
    out_type=jax.ShapeDtypeStruct((_N_TOTAL * _DIM,), jnp.float32),
    scratch_types=[
        pltpu.VMEM((_CHUNK,), jnp.int32),
        pltpu.VMEM((_CHUNK,), jnp.int32),
        pltpu.VMEM((_CHUNK * _DIM,), jnp.float32),
        pltpu.VMEM((_CHUNK * _DIM,), jnp.float32),
        pltpu.VMEM((_VOCAB * _DIM,), jnp.float32),
        pltpu.SemaphoreType.DMA,
        pltpu.SemaphoreType.DMA,
        pltpu.SemaphoreType.DMA,
        pltpu.SemaphoreType.DMA,
    ],
    compiler_params=pltpu.CompilerParams(use_tc_tiling_on_sc=False,
                                         needs_layout_passes=False),
)
def _gather_kernel(idx_hbm, table_hbm, out_hbm,
                   idx0, idx1, rows0, rows1, table_v,
                   sl0, sl1, ss0, ss1):
    sid = lax.axis_index("s")
    wid = sid * _NUM_CORES + lax.axis_index("c")
    base = wid * _B_PER_W

    idx = (idx0, idx1)
    rows = (rows0, rows1)
    sl = (sl0, sl1)
    ss = (ss0, ss1)

    pltpu.sync_copy(table_hbm, table_v)

    lane = lax.iota(jnp.int32, _LANES)
    lane32 = lane * _DIM

    def issue_l(i, b):
        pltpu.async_copy(idx_hbm.at[pl.ds(base + i * _CHUNK, _CHUNK)],
                         idx[b], sl[b])

    def wait_l(b):
        pltpu.make_async_copy(idx_hbm.at[pl.ds(base, _CHUNK)],
                              idx[b], sl[b]).wait()

    def issue_s(i, b):
        pltpu.async_copy(
            rows[b],
            out_hbm.at[pl.ds((base + i * _CHUNK) * _DIM, _CHUNK * _DIM)],
            ss[b])

    def wait_s(b):
        pltpu.make_async_copy(rows[b],
                              out_hbm.at[pl.ds(base * _DIM, _CHUNK * _DIM)],
                              ss[b]).wait()

    def compute(b):
        idx_ref = idx[b]
        rows_ref = rows[b]

        @plsc.parallel_loop(0, _GROUPS, unroll=4)
        def group(g):
            idx16 = idx_ref[pl.ds(g * _LANES, _LANES)]
            src = idx16 * _DIM
            dst = g * (_LANES * _DIM) + lane32
            for d in range(_DIM):
                vals = plsc.load_gather(table_v, [src + d])
                plsc.store_scatter(rows_ref, [dst + d], vals)

    issue_l(0, 0)

    def step(i, b, ob):
        @pl.when(i >= 2)
        def _():
            wait_s(b)

        wait_l(b)

        @pl.when(i + 1 < _N_CHUNKS)
        def _():
            issue_l(i + 1, ob)

        compute(b)
        issue_s(i, b)

    def outer(g, carry):
        step(2 * g, 0, 1)
        step(2 * g + 1, 1, 0)
        return carry

    lax.fori_loop(0, _N_CHUNKS // 2, outer, 0)

    wait_s(0)
    wait_s(1)


def kernel(sentence, table):
    flat_idx = sentence.reshape(_N_TOTAL)
    out = _gather_kernel(flat_idx, table.reshape(_VOCAB * _DIM))
    return out.reshape(_BATCH, _SEQ, _DIM)


# speedup vs baseline: 2.3955x; 1.9813x over previous
import functools

import jax
import jax.numpy as jnp
from jax import lax
from jax.experimental import pallas as pl
from jax.experimental.pallas import tpu as pltpu
from jax.experimental.pallas import tpu_sc as plsc

_BATCH = 16384
_SEQ = 200
_DIM = 32
_VOCAB = 1000
_N_TOTAL = _BATCH * _SEQ
_NUM_CORES = 2
_NUM_SUBCORES = 16
_NW = _NUM_CORES * _NUM_SUBCORES
_B_PER_W = _N_TOTAL // _NW
_CHUNK = 1600
_N_CHUNKS = _B_PER_W // _CHUNK
_SP = 800
_HB = _CHUNK - _SP

_mesh = plsc.VectorSubcoreMesh(core_axis_name="c", subcore_axis_name="s")


@functools.partial(
    pl.kernel,
    mesh=_mesh,
    out_type=jax.ShapeDtypeStruct((_N_TOTAL, _DIM), jnp.float32),
    scratch_types=[
        pltpu.VMEM((_CHUNK,), jnp.int32),
        pltpu.VMEM((_CHUNK,), jnp.int32),
        pltpu.VMEM((_CHUNK, _DIM), jnp.float32),
        pltpu.VMEM((_CHUNK, _DIM), jnp.float32),
        pltpu.VMEM_SHARED((_VOCAB, _DIM), jnp.float32),
        pltpu.SemaphoreType.DMA,
        pltpu.SemaphoreType.DMA,
        pltpu.SemaphoreType.DMA,
        pltpu.SemaphoreType.DMA,
        pltpu.SemaphoreType.DMA,
        pltpu.SemaphoreType.DMA,
        pltpu.SemaphoreType.DMA,
        pltpu.SemaphoreType.DMA,
    ],
    compiler_params=pltpu.CompilerParams(use_tc_tiling_on_sc=False),
)
def _gather_kernel(idx_hbm, table_hbm, out_hbm,
                   idx0, idx1, rows0, rows1, table_v,
                   sl0, sl1, sg0, sg1, sh0, sh1, ss0, ss1):
    sid = lax.axis_index("s")
    wid = sid * _NUM_CORES + lax.axis_index("c")
    base = wid * _B_PER_W

    idx = (idx0, idx1)
    rows = (rows0, rows1)
    sl = (sl0, sl1)
    sg = (sg0, sg1)
    sh = (sh0, sh1)
    ss = (ss0, ss1)

    @pl.when(sid == 0)
    def _():
        pltpu.sync_copy(table_hbm, table_v)

    plsc.subcore_barrier()

    def issue_l(i, b):
        pltpu.async_copy(idx_hbm.at[pl.ds(base + i * _CHUNK, _CHUNK)],
                         idx[b], sl[b])

    def wait_l(b):
        pltpu.make_async_copy(idx_hbm.at[pl.ds(base, _CHUNK)],
                              idx[b], sl[b]).wait()

    def issue_g(b):
        pltpu.async_copy(table_v.at[idx[b].at[pl.ds(0, _SP)]],
                         rows[b].at[pl.ds(0, _SP)], sg[b])
        pltpu.async_copy(table_hbm.at[idx[b].at[pl.ds(_SP, _HB)]],
                         rows[b].at[pl.ds(_SP, _HB)], sh[b])

    def wait_g(b):
        pltpu.make_async_copy(table_v.at[idx[b].at[pl.ds(0, _SP)]],
                              rows[b].at[pl.ds(0, _SP)], sg[b]).wait()
        pltpu.make_async_copy(table_hbm.at[idx[b].at[pl.ds(_SP, _HB)]],
                              rows[b].at[pl.ds(_SP, _HB)], sh[b]).wait()

    def issue_s(i, b):
        pltpu.async_copy(rows[b], out_hbm.at[pl.ds(base + i * _CHUNK, _CHUNK)],
                         ss[b])

    def wait_s(b):
        pltpu.make_async_copy(rows[b], out_hbm.at[pl.ds(base, _CHUNK)],
                              ss[b]).wait()

    issue_l(0, 0)

    def step(i, b, ob):
        @pl.when(i >= 2)
        def _():
            wait_s(b)

        wait_l(b)
        issue_g(b)

        @pl.when(i >= 1)
        def _():
            wait_g(ob)
            issue_s(i - 1, ob)

        @pl.when(i + 1 < _N_CHUNKS)
        def _():
            issue_l(i + 1, ob)

    def outer(g, carry):
        step(2 * g, 0, 1)
        step(2 * g + 1, 1, 0)
        return carry

    lax.fori_loop(0, _N_CHUNKS // 2, outer, 0)

    wait_g(1)
    issue_s(_N_CHUNKS - 1, 1)
    wait_s(0)
    wait_s(1)


def kernel(sentence, table):
    flat_idx = sentence.reshape(_N_TOTAL)
    out = _gather_kernel(flat_idx, table)
    return out.reshape(_BATCH, _SEQ, _DIM)


# speedup vs baseline: 2.9639x; 1.2373x over previous
import functools

import jax
import jax.numpy as jnp
from jax import lax
from jax.experimental import pallas as pl
from jax.experimental.pallas import tpu as pltpu
from jax.experimental.pallas import tpu_sc as plsc

_BATCH = 16384
_SEQ = 200
_DIM = 32
_VOCAB = 1000
_N_TOTAL = _BATCH * _SEQ
_NUM_CORES = 2
_NUM_SUBCORES = 16
_NW = _NUM_CORES * _NUM_SUBCORES
_B_PER_W = _N_TOTAL // _NW
_CHUNK = 1280
_N_CHUNKS = _B_PER_W // _CHUNK
_LANES = 16

_mesh = plsc.VectorSubcoreMesh(core_axis_name="c", subcore_axis_name="s")


@functools.partial(
    pl.kernel,
    mesh=_mesh,
    out_type=jax.ShapeDtypeStruct((_N_TOTAL * _DIM,), jnp.float32),
    scratch_types=[
        pltpu.VMEM((_CHUNK,), jnp.int32),
        pltpu.VMEM((_CHUNK,), jnp.int32),
        pltpu.VMEM((_CHUNK * _DIM,), jnp.float32),
        pltpu.VMEM((_CHUNK * _DIM,), jnp.float32),
        pltpu.VMEM((_VOCAB * _DIM,), jnp.float32),
        pltpu.SemaphoreType.DMA,
        pltpu.SemaphoreType.DMA,
        pltpu.SemaphoreType.DMA,
        pltpu.SemaphoreType.DMA,
    ],
    compiler_params=pltpu.CompilerParams(use_tc_tiling_on_sc=False,
                                         needs_layout_passes=False),
)
def _gather_kernel(idx_hbm, table_hbm, out_hbm,
                   idx0, idx1, rows0, rows1, table_v,
                   sl0, sl1, ss0, ss1):
    sid = lax.axis_index("s")
    wid = sid * _NUM_CORES + lax.axis_index("c")
    base = wid * _B_PER_W

    idx = (idx0, idx1)
    rows = (rows0, rows1)
    sl = (sl0, sl1)
    ss = (ss0, ss1)

    pltpu.sync_copy(table_hbm, table_v)

    def issue_l(i, b):
        pltpu.async_copy(idx_hbm.at[pl.ds(base + i * _CHUNK, _CHUNK)],
                         idx[b], sl[b])

    def wait_l(b):
        pltpu.make_async_copy(idx_hbm.at[pl.ds(base, _CHUNK)],
                              idx[b], sl[b]).wait()

    def issue_s(i, b):
        pltpu.async_copy(
            rows[b],
            out_hbm.at[pl.ds((base + i * _CHUNK) * _DIM, _CHUNK * _DIM)],
            ss[b])

    def wait_s(b):
        pltpu.make_async_copy(rows[b],
                              out_hbm.at[pl.ds(base * _DIM, _CHUNK * _DIM)],
                              ss[b]).wait()

    def compute(b):
        idx_ref = idx[b]
        rows_ref = rows[b]

        @plsc.parallel_loop(0, _CHUNK // _LANES, unroll=2)
        def group(g):
            iv = idx_ref[pl.ds(g * _LANES, _LANES)] * _DIM
            gbase = g * (_LANES * _DIM)
            for l in range(_LANES):
                off = iv[l]
                dst = gbase + l * _DIM
                rows_ref[pl.ds(dst, _LANES)] = table_v[pl.ds(off, _LANES)]
                rows_ref[pl.ds(dst + _LANES, _LANES)] = (
                    table_v[pl.ds(off + _LANES, _LANES)])

    issue_l(0, 0)

    def step(i, b, ob):
        @pl.when(i >= 2)
        def _():
            wait_s(b)

        wait_l(b)

        @pl.when(i + 1 < _N_CHUNKS)
        def _():
            issue_l(i + 1, ob)

        compute(b)
        issue_s(i, b)

    def outer(g, carry):
        step(2 * g, 0, 1)
        step(2 * g + 1, 1, 0)
        return carry

    lax.fori_loop(0, _N_CHUNKS // 2, outer, 0)

    wait_s(0)
    wait_s(1)


def kernel(sentence, table):
    flat_idx = sentence.reshape(_N_TOTAL)
    out = _gather_kernel(flat_idx, table.reshape(_VOCAB * _DIM))
    return out.reshape(_BATCH, _SEQ, _DIM)
